# baseline (speedup 1.0000x reference)
import functools

import jax
import jax.numpy as jnp
from jax.experimental import pallas as pl
from jax.experimental.pallas import tpu as pltpu

BETA = 5.0
XY_WEIGHT = 0.5
B, M, DSLOT, C, K, D = 64, 256, 126, 100, 32, 128
CK = C * K
LOG2E = 1.4426950408889634
NB = 8


def _mixer_kernel(tp_ref, ap_ref, s_ref, xy_ref, p_row_ref, m_row_ref,
                  p_col_ref, m_col_ref, base_ref, cf_ref, psif_ref, out_ref,
                  cft_scr, cb_scr, sel_scr):
    pid = pl.program_id(0)

    @pl.when(pid == 0)
    def _prep():
        cf = cf_ref[...]
        cft_scr[...] = cf.T
        cft = cft_scr[...]
        c2 = jnp.dot(jnp.ones((1, D), jnp.float32), cft * cft,
                     preferred_element_type=jnp.float32)
        seg = jax.lax.broadcasted_iota(jnp.int32, (CK, D), 0) // K
        cidx = jax.lax.broadcasted_iota(jnp.int32, (CK, D), 1)
        sel = (seg == cidx).astype(jnp.float32)
        sel_scr[...] = sel
        psif = psif_ref[...]
        mg = jnp.max(psif)
        e = jnp.exp(psif - mg)
        seg_sum = jnp.dot(e, sel, preferred_element_type=jnp.float32)
        sums = jax.lax.dot_general(seg_sum, sel, (((1,), (1,)), ((), ())),
                                   preferred_element_type=jnp.float32)
        lnw = psif - mg - jnp.log(sums)
        cb_scr[...] = LOG2E * (lnw - BETA * c2)

    MM = NB * M
    s_in = s_ref[...].reshape(MM, DSLOT)
    xy_in = xy_ref[...].reshape(MM, 2)
    sxy = jnp.concatenate([s_in, xy_in], axis=-1)
    xsq = sxy * sxy
    lane = jax.lax.broadcasted_iota(jnp.int32, (MM, D), 1)
    is_s = lane < DSLOT
    n1 = jnp.sqrt(jnp.sum(jnp.where(is_s, xsq, 0.0), axis=1, keepdims=True))
    n2 = jnp.sqrt(jnp.sum(jnp.where(is_s, 0.0, xsq), axis=1, keepdims=True))
    scale = jnp.where(is_s,
                      1.0 / jnp.maximum(n1, 1e-12),
                      XY_WEIGHT / jnp.maximum(n2, 1e-12))
    a = sxy * scale
    a2 = a * a

    s_row = p_row_ref[...] * m_row_ref[...]
    s_col = p_col_ref[...] * m_col_ref[...]
    idx_col = jax.lax.broadcasted_iota(jnp.int32, (NB, M, 1), 1)
    idx_row = jax.lax.broadcasted_iota(jnp.int32, (NB, 1, M), 2)
    before = (s_col > s_row) | ((s_col == s_row) & (idx_col <= idx_row))
    beforef = before.astype(jnp.float32)
    csum = jnp.sum(s_col * beforef, axis=1, keepdims=True)
    rank = jnp.sum(beforef, axis=1, keepdims=True) - 1.0
    total = jnp.sum(s_row, axis=2, keepdims=True)
    thresh = tp_ref[0, 0] * (total + 1e-8)
    cnt = jnp.sum((csum <= thresh).astype(jnp.float32), axis=2,
                  keepdims=True)
    k = jnp.maximum(cnt, 1.0)
    wm = jnp.where(rank < k, 1.0 / k, 0.0).reshape(NB, M)
    ones_row = jnp.ones((1, D), jnp.float32)
    s2_rows = [jax.lax.dot_general(ones_row, a2[i * M:(i + 1) * M, :],
                                   (((1,), (1,)), ((), ())),
                                   preferred_element_type=jnp.float32)
               for i in range(NB)]
    s2_row = jnp.concatenate(s2_rows, axis=0)
    wm = wm * jnp.exp2((-BETA * LOG2E) * s2_row)

    g = jnp.dot(a * (2.0 * BETA * LOG2E), cft_scr[...],
                preferred_element_type=jnp.float32)
    sim = jnp.exp2(g + cb_scr[...]).astype(jnp.bfloat16)
    wmb = wm.astype(jnp.bfloat16)
    ts = [jnp.dot(wmb[i:i + 1, :], sim[i * M:(i + 1) * M, :],
                  preferred_element_type=jnp.float32)
          for i in range(NB)]
    t = jnp.concatenate(ts, axis=0)
    scores = jnp.dot(t, sel_scr[...],
                     preferred_element_type=jnp.float32)
    alpha = jax.nn.sigmoid(ap_ref[0, 0])
    out_ref[...] = (alpha * base_ref[...]
                    + (1.0 - alpha) * scores[:, 0:C].reshape(NB, 1, C))


@jax.jit
def kernel(base_b, S_slots_b, XY_b, P_b, mask_b, centers, psi, alpha_param,
           top_p):
    f32 = jnp.float32
    cf = centers.reshape(CK, D)
    psif = psi.reshape(1, CK)
    p2 = P_b.reshape(B, 1, M)
    m2 = mask_b.reshape(B, 1, M)
    p3 = P_b[..., None]
    m3 = mask_b[..., None]
    base3 = base_b.reshape(B, 1, C)
    tp = jnp.reshape(top_p.astype(f32), (1, 1))
    ap = jnp.reshape(alpha_param.astype(f32), (1, 1))

    grid = (B // NB,)
    fixed = lambda i: (0, 0)
    out = pl.pallas_call(
        _mixer_kernel,
        grid=grid,
        in_specs=[
            pl.BlockSpec((1, 1), fixed),
            pl.BlockSpec((1, 1), fixed),
            pl.BlockSpec((NB, M, DSLOT), lambda i: (i, 0, 0)),
            pl.BlockSpec((NB, M, 2), lambda i: (i, 0, 0)),
            pl.BlockSpec((NB, 1, M), lambda i: (i, 0, 0)),
            pl.BlockSpec((NB, 1, M), lambda i: (i, 0, 0)),
            pl.BlockSpec((NB, M, 1), lambda i: (i, 0, 0)),
            pl.BlockSpec((NB, M, 1), lambda i: (i, 0, 0)),
            pl.BlockSpec((NB, 1, C), lambda i: (i, 0, 0)),
            pl.BlockSpec((CK, D), fixed),
            pl.BlockSpec((1, CK), fixed),
        ],
        out_specs=pl.BlockSpec((NB, 1, C), lambda i: (i, 0, 0)),
        out_shape=jax.ShapeDtypeStruct((B, 1, C), f32),
        scratch_shapes=[
            pltpu.VMEM((D, CK), f32),
            pltpu.VMEM((1, CK), f32),
            pltpu.VMEM((CK, D), f32),
        ],
    )(tp, ap, S_slots_b, XY_b, p2, m2, p3, m3, base3, cf, psif)
    return out.reshape(B, C)
